# baseline (speedup 1.0000x reference)
import jax
import jax.numpy as jnp
from jax.experimental import pallas as pl
from jax.experimental.pallas import tpu as pltpu

BATCH = 10000
DIM = 512
NUM_CHUNKS = 5
ROWS_PER_CHUNK = BATCH // NUM_CHUNKS


def _dma_body(x_hbm, o_hbm, sems):
    copies = [
        pltpu.make_async_copy(
            x_hbm.at[pl.ds(i * ROWS_PER_CHUNK, ROWS_PER_CHUNK), :],
            o_hbm.at[pl.ds(i * ROWS_PER_CHUNK, ROWS_PER_CHUNK), :],
            sems.at[i],
        )
        for i in range(NUM_CHUNKS)
    ]
    for c in copies:
        c.start()
    for c in copies:
        c.wait()


def kernel(x, ind, mask, sampled, embed):
    del ind, mask, sampled, embed
    return pl.pallas_call(
        _dma_body,
        in_specs=[pl.BlockSpec(memory_space=pltpu.MemorySpace.HBM)],
        out_specs=pl.BlockSpec(memory_space=pltpu.MemorySpace.HBM),
        out_shape=jax.ShapeDtypeStruct((BATCH, DIM), jnp.float32),
        scratch_shapes=[pltpu.SemaphoreType.DMA((NUM_CHUNKS,))],
    )(x)


# speedup vs baseline: 39.3840x; 39.3840x over previous
import jax
import jax.numpy as jnp
from jax.experimental import pallas as pl

BATCH = 10000
DIM = 512
ROWS_PER_BLOCK = 1000


def _copy_body(x_ref, o_ref):
    o_ref[...] = x_ref[...]


def kernel(x, ind, mask, sampled, embed):
    del ind, mask, sampled, embed
    return pl.pallas_call(
        _copy_body,
        grid=(BATCH // ROWS_PER_BLOCK,),
        in_specs=[pl.BlockSpec((ROWS_PER_BLOCK, DIM), lambda i: (i, 0))],
        out_specs=pl.BlockSpec((ROWS_PER_BLOCK, DIM), lambda i: (i, 0)),
        out_shape=jax.ShapeDtypeStruct((BATCH, DIM), jnp.float32),
    )(x)


# speedup vs baseline: 48.6542x; 1.2354x over previous
import jax
import jax.numpy as jnp
from jax.experimental import pallas as pl

BATCH = 10000
DIM = 512
ROWS_PER_BLOCK = 5000


def _copy_body(x_ref, o_ref):
    o_ref[...] = x_ref[...]


def kernel(x, ind, mask, sampled, embed):
    del ind, mask, sampled, embed
    return pl.pallas_call(
        _copy_body,
        grid=(BATCH // ROWS_PER_BLOCK,),
        in_specs=[pl.BlockSpec((ROWS_PER_BLOCK, DIM), lambda i: (i, 0))],
        out_specs=pl.BlockSpec((ROWS_PER_BLOCK, DIM), lambda i: (i, 0)),
        out_shape=jax.ShapeDtypeStruct((BATCH, DIM), jnp.float32),
    )(x)
